# baseline (speedup 1.0000x reference)
import jax
import jax.numpy as jnp
from jax import lax
from jax.experimental import pallas as pl
from jax.experimental.pallas import tpu as pltpu
from jax.experimental.pallas import tpu_sc as plsc

_B = 16384
_D = 64
_NC = 2
_NS = 16
_NW = _NC * _NS
_BPW = _B // _NW
_NCH = 4
_CH = _BPW // _NCH
_NG = _BPW // 16


def _rsqrt(x):
    i = lax.bitcast_convert_type(x, jnp.int32)
    seed = jnp.int32(0x5F3759DF) - lax.shift_right_logical(i, 1)
    y = lax.bitcast_convert_type(seed, jnp.float32)
    for _ in range(3):
        y = y * (1.5 - 0.5 * x * y * y)
    return y


def _body(ph, pr, pt, nh, nr, nt, ent, rel, p_out, n_out,
          ia, ib, ic, abuf, bbuf, cbuf, obuf, sem):
    wid = lax.axis_index("s") * _NC + lax.axis_index("c")
    iot = lax.iota(jnp.int32, 16)

    for idx_a, tab_a, idx_b, tab_b, idx_c, tab_c, out in (
        (ph, ent, pr, rel, pt, ent, p_out),
        (nh, ent, nt, ent, nr, rel, n_out),
    ):
        row0 = wid * _NCH
        pltpu.sync_copy(idx_a.at[pl.ds(row0, _NCH)], ia)
        pltpu.sync_copy(idx_b.at[pl.ds(row0, _NCH)], ib)
        pltpu.sync_copy(idx_c.at[pl.ds(row0, _NCH)], ic)
        dmas = []
        for c in range(_NCH):
            sl = pl.ds(c * _CH, _CH)
            dmas.append(pltpu.async_copy(tab_a.at[ia.at[c]], abuf.at[sl], sem))
            dmas.append(pltpu.async_copy(tab_b.at[ib.at[c]], bbuf.at[sl], sem))
            dmas.append(pltpu.async_copy(tab_c.at[ic.at[c]], cbuf.at[sl], sem))
        for d in dmas:
            d.wait()

        def group(g, carry):
            r = g * 16 + iot
            z = jnp.zeros((16,), jnp.float32)
            aa, bb, cc, ab, ac, bc = z, z, z, z, z, z
            for d in range(_D):
                col = jnp.bitwise_and(iot + d, _D - 1)
                av = plsc.load_gather(abuf, [r, col])
                bv = plsc.load_gather(bbuf, [r, col])
                cv = plsc.load_gather(cbuf, [r, col])
                aa += av * av
                bb += bv * bv
                cc += cv * cv
                ab += av * bv
                ac += av * cv
                bc += bv * cv
            inva = _rsqrt(jnp.maximum(aa, 1e-24))
            invb = _rsqrt(jnp.maximum(bb, 1e-24))
            invc = _rsqrt(jnp.maximum(cc, 1e-24))
            s2 = 3.0 + 2.0 * (ab * inva * invb - ac * inva * invc
                              - bc * invb * invc)
            s2 = jnp.maximum(s2, 0.0)
            score = s2 * _rsqrt(jnp.maximum(s2, 1e-30))
            obuf[pl.ds(g * 16, 16)] = score
            return carry

        lax.fori_loop(0, _NG, group, 0)
        pltpu.sync_copy(obuf, out.at[pl.ds(wid * _BPW, _BPW)])


def kernel(pos_h, pos_r, pos_t, neg_h, neg_r, neg_t, ent_emb, rel_emb):
    shp = (_B // _CH, _CH)
    ph = pos_h.astype(jnp.int32).reshape(shp)
    pr = pos_r.astype(jnp.int32).reshape(shp)
    pt = pos_t.astype(jnp.int32).reshape(shp)
    nh = neg_h.astype(jnp.int32).reshape(shp)
    nr = neg_r.astype(jnp.int32).reshape(shp)
    nt = neg_t.astype(jnp.int32).reshape(shp)

    mesh = plsc.VectorSubcoreMesh(core_axis_name="c", subcore_axis_name="s")
    run = pl.kernel(
        _body,
        mesh=mesh,
        compiler_params=pltpu.CompilerParams(
            use_tc_tiling_on_sc=False, needs_layout_passes=False
        ),
        out_type=[
            jax.ShapeDtypeStruct((_B,), jnp.float32),
            jax.ShapeDtypeStruct((_B,), jnp.float32),
        ],
        scratch_types=[
            pltpu.VMEM((_NCH, _CH), jnp.int32),
            pltpu.VMEM((_NCH, _CH), jnp.int32),
            pltpu.VMEM((_NCH, _CH), jnp.int32),
            pltpu.VMEM((_BPW, _D), jnp.float32),
            pltpu.VMEM((_BPW, _D), jnp.float32),
            pltpu.VMEM((_BPW, _D), jnp.float32),
            pltpu.VMEM((_BPW,), jnp.float32),
            pltpu.SemaphoreType.DMA,
        ],
    )
    p_score, n_score = run(ph, pr, pt, nh, nr, nt, ent_emb, rel_emb)
    return (p_score, n_score)
